# baseline (speedup 1.0000x reference)
import functools

import jax
import jax.numpy as jnp
from jax import lax
from jax.experimental import pallas as pl

B, L, D = 256, 128, 6
F_ATOM, F_BOND, FP = 39, 10, 128


def _lrelu(x):
    return jnp.maximum(x, 0.01 * x)


def _dgT(a, b):
    return lax.dot_general(a, b, (((0,), (0,)), ((), ())),
                           preferred_element_type=jnp.float32)


M = 8


def _fused_kernel(atomT_ref, bondT_ref, aidxT_ref, bidxT_ref,
                  w_atom2_ref, wnb_a2_ref, wnb_b2_ref,
                  w1_ref, w2_ref, b_align_ref, w_att_ref, b_att_ref,
                  w_ihT_ref, b_ih_ref, w_hhT_ref, b_hh_ref,
                  out_ref):
    dot = functools.partial(jnp.dot, preferred_element_type=jnp.float32)
    bf = jnp.bfloat16
    iota_b = lax.broadcasted_iota(jnp.int32, (L, L), 0).astype(bf)
    w2 = w2_ref[...]
    ML = M * L

    atomsT_all = jnp.concatenate([atomT_ref[m] for m in range(M)], axis=1)
    bondsT_all = jnp.concatenate([bondT_ref[m] for m in range(M)], axis=1)
    afT_all = _lrelu(_dgT(w_atom2_ref[...], atomsT_all))
    af16_all = afT_all.astype(bf)
    align1_all = _dgT(w1_ref[...], af16_all) + b_align_ref[0, 0]
    apT_all = _dgT(wnb_a2_ref[...], atomsT_all).astype(bf)
    bpT_all = _dgT(wnb_b2_ref[...], bondsT_all).astype(bf)

    nf_alls, score_alls, valids = [], [], []
    for m in range(M):
        gacols, gbcols = [], []
        for d in range(D):
            arow = aidxT_ref[m, d:d + 1, :]
            brow = bidxT_ref[m, d:d + 1, :]
            da = arow - iota_b
            db = brow - iota_b
            gacols.append(jnp.maximum(1.0 - da * da, 0.0))
            gbcols.append(jnp.maximum(1.0 - db * db, 0.0))
        g2 = jnp.concatenate([jnp.concatenate(gacols, axis=1),
                              jnp.concatenate(gbcols, axis=1)],
                             axis=0)
        tp = jnp.concatenate([apT_all[:, m * L:(m + 1) * L],
                              bpT_all[:, m * L:(m + 1) * L]], axis=1)
        nf16 = _lrelu(dot(tp, g2).astype(bf))
        sc_all = _dgT(w2, nf16)
        nf_alls.append(nf16)
        score_alls.append(sc_all)
        valids.append((aidxT_ref[m] != L - 1).astype(jnp.float32))

    accs, tots = [], []
    for m in range(M):
        al = align1_all[:, m * L:(m + 1) * L]
        scores = []
        for d in range(D):
            s = _lrelu(al + score_alls[m][:, d * L:(d + 1) * L])
            v = valids[m][d:d + 1, :]
            scores.append(jnp.where(v > 0, s, s - 9e8))
        smax = scores[0]
        for d in range(1, D):
            smax = jnp.maximum(smax, scores[d])
        exps = [jnp.exp(s - smax) for s in scores]
        denom = exps[0]
        for d in range(1, D):
            denom = denom + exps[d]
        inv = 1.0 / denom
        acc = None
        tot = None
        for d in range(D):
            attn = exps[d] * inv * valids[m][d:d + 1, :]
            term = attn.astype(bf) * nf_alls[m][:, d * L:(d + 1) * L]
            acc = term if acc is None else acc + term
            tot = attn if tot is None else tot + attn
        accs.append(acc)
        tots.append(tot)
    acc_all = jnp.concatenate(accs, axis=1)
    tot_all = jnp.concatenate(tots, axis=1)
    ctx_pre = (_dgT(w_att_ref[...], acc_all)
               + b_att_ref[...] * tot_all)
    ctx16 = jnp.where(ctx_pre > 0, ctx_pre,
                      jnp.exp(ctx_pre) - 1.0).astype(bf)

    x2 = jnp.concatenate([ctx16, af16_all], axis=0)
    p = _dgT(x2, w_ihT_ref[...]) + b_ih_ref[...]
    ghn = _dgT(af16_all, w_hhT_ref[...]) + b_hh_ref[...]
    r = jax.nn.sigmoid(p[:, :FP])
    z = jax.nn.sigmoid(p[:, FP:2 * FP])
    n = jnp.tanh(p[:, 2 * FP:] - (1.0 - r) * ghn)
    af_nat = afT_all.T
    hnew = (1.0 - z) * n + z * af_nat
    out_ref[...] = jnp.maximum(hnew, 0.0).reshape(M, L, FP)


def kernel(atom_list, bond_list, atom_degree_list, bond_degree_list, atom_mask,
           W_atom, b_atom, W_nb, b_nb, W_align, b_align, W_att, b_att,
           W_ih, W_hh, b_ih, b_hh):
    del atom_mask
    ones = jnp.ones((B, 1, L), jnp.float32)
    bf = jnp.bfloat16
    atomsT2 = jnp.concatenate([atom_list.transpose(0, 2, 1), ones], axis=1).astype(bf)
    bondsT2 = jnp.concatenate([bond_list.transpose(0, 2, 1), ones], axis=1).astype(bf)
    aidxT = atom_degree_list.astype(jnp.int32).transpose(0, 2, 1).astype(bf)
    bidxT = bond_degree_list.astype(jnp.int32).transpose(0, 2, 1).astype(bf)

    w_atom2 = jnp.concatenate([W_atom, b_atom[None, :]], axis=0).astype(bf)
    half_bnb = 0.5 * b_nb[None, :]
    wnb_a2 = jnp.concatenate([W_nb[:F_ATOM], half_bnb], axis=0).astype(bf)
    wnb_b2 = jnp.concatenate([W_nb[F_ATOM:], half_bnb], axis=0).astype(bf)
    w1 = W_align[:FP].astype(bf)
    w2 = W_align[FP:].astype(bf)
    b_align2 = b_align.reshape(1, 1)
    b_att_col = b_att.reshape(FP, 1)
    w_att16 = W_att.astype(bf)
    w_ihT = jnp.concatenate([W_ih.T, W_hh.T], axis=0).astype(bf)
    b_ih_sum = b_ih + b_hh
    w_hhT = W_hh.T[:, 2 * FP:].astype(bf)
    b_hh_n = b_hh[2 * FP:]

    rep = lambda arr: pl.BlockSpec(arr.shape, lambda i: (0,) * arr.ndim)
    row = lambda v: v.reshape(1, -1)

    out = pl.pallas_call(
        _fused_kernel,
        grid=(B // M,),
        in_specs=[
            pl.BlockSpec((M, F_ATOM + 1, L), lambda i: (i, 0, 0)),
            pl.BlockSpec((M, F_BOND + 1, L), lambda i: (i, 0, 0)),
            pl.BlockSpec((M, D, L), lambda i: (i, 0, 0)),
            pl.BlockSpec((M, D, L), lambda i: (i, 0, 0)),
            rep(w_atom2), rep(wnb_a2), rep(wnb_b2),
            rep(w1), rep(w2), rep(b_align2),
            rep(w_att16), rep(b_att_col),
            rep(w_ihT), rep(row(b_ih_sum)),
            rep(w_hhT), rep(row(b_hh_n)),
        ],
        out_specs=pl.BlockSpec((M, L, FP), lambda i: (i, 0, 0)),
        out_shape=jax.ShapeDtypeStruct((B, L, FP), jnp.float32),
    )(atomsT2, bondsT2, aidxT, bidxT,
      w_atom2, wnb_a2, wnb_b2, w1, w2, b_align2, w_att16, b_att_col,
      w_ihT, row(b_ih_sum), w_hhT, row(b_hh_n))
    return out


# speedup vs baseline: 1.0007x; 1.0007x over previous
import functools

import jax
import jax.numpy as jnp
from jax import lax
from jax.experimental import pallas as pl
from jax.experimental.pallas import tpu as pltpu

B, L, D = 256, 128, 6
F_ATOM, F_BOND, FP = 39, 10, 128


def _lrelu(x):
    return jnp.maximum(x, 0.01 * x)


def _dgT(a, b):
    return lax.dot_general(a, b, (((0,), (0,)), ((), ())),
                           preferred_element_type=jnp.float32)


M = 8


def _fused_kernel(atomT_ref, bondT_ref, aidxT_ref, bidxT_ref,
                  w_atom2_ref, wnb_a2_ref, wnb_b2_ref,
                  w1_ref, w2_ref, b_align_ref, w_att_ref, b_att_ref,
                  w_ihT_ref, b_ih_ref, w_hhT_ref, b_hh_ref,
                  out_ref):
    dot = functools.partial(jnp.dot, preferred_element_type=jnp.float32)
    bf = jnp.bfloat16
    iota_b = lax.broadcasted_iota(jnp.int32, (L, L), 0).astype(bf)
    w2 = w2_ref[...]
    ML = M * L

    atomsT_all = jnp.concatenate([atomT_ref[m] for m in range(M)], axis=1)
    bondsT_all = jnp.concatenate([bondT_ref[m] for m in range(M)], axis=1)
    afT_all = _lrelu(_dgT(w_atom2_ref[...], atomsT_all))
    af16_all = afT_all.astype(bf)
    align1_all = _dgT(w1_ref[...], af16_all) + b_align_ref[0, 0]
    apT_all = _dgT(wnb_a2_ref[...], atomsT_all).astype(bf)
    bpT_all = _dgT(wnb_b2_ref[...], bondsT_all).astype(bf)

    nf_alls, score_alls, valids = [], [], []
    for m in range(M):
        gacols, gbcols = [], []
        for d in range(D):
            arow = aidxT_ref[m, d:d + 1, :]
            brow = bidxT_ref[m, d:d + 1, :]
            da = arow - iota_b
            db = brow - iota_b
            gacols.append(jnp.maximum(1.0 - da * da, 0.0))
            gbcols.append(jnp.maximum(1.0 - db * db, 0.0))
        g2 = jnp.concatenate([jnp.concatenate(gacols, axis=1),
                              jnp.concatenate(gbcols, axis=1)],
                             axis=0)
        tp = jnp.concatenate([apT_all[:, m * L:(m + 1) * L],
                              bpT_all[:, m * L:(m + 1) * L]], axis=1)
        nf16 = _lrelu(dot(tp, g2).astype(bf))
        sc_all = _dgT(w2, nf16)
        nf_alls.append(nf16)
        score_alls.append(sc_all)
        valids.append((aidxT_ref[m] != L - 1).astype(jnp.float32))

    accs, tots = [], []
    for m in range(M):
        al = align1_all[:, m * L:(m + 1) * L]
        scores = []
        for d in range(D):
            s = _lrelu(al + score_alls[m][:, d * L:(d + 1) * L])
            v = valids[m][d:d + 1, :]
            scores.append(jnp.where(v > 0, s, s - 9e8))
        smax = scores[0]
        for d in range(1, D):
            smax = jnp.maximum(smax, scores[d])
        exps = [jnp.exp(s - smax) for s in scores]
        denom = exps[0]
        for d in range(1, D):
            denom = denom + exps[d]
        inv = 1.0 / denom
        acc = None
        tot = None
        for d in range(D):
            attn = exps[d] * inv * valids[m][d:d + 1, :]
            term = attn.astype(bf) * nf_alls[m][:, d * L:(d + 1) * L]
            acc = term if acc is None else acc + term
            tot = attn if tot is None else tot + attn
        accs.append(acc)
        tots.append(tot)
    acc_all = jnp.concatenate(accs, axis=1)
    tot_all = jnp.concatenate(tots, axis=1)
    ctx_pre = (_dgT(w_att_ref[...], acc_all)
               + b_att_ref[...] * tot_all)
    ctx16 = jnp.where(ctx_pre > 0, ctx_pre,
                      jnp.exp(ctx_pre) - 1.0).astype(bf)

    x2 = jnp.concatenate([ctx16, af16_all], axis=0)
    p = _dgT(x2, w_ihT_ref[...]) + b_ih_ref[...]
    ghn = _dgT(af16_all, w_hhT_ref[...]) + b_hh_ref[...]
    r = jax.nn.sigmoid(p[:, :FP])
    z = jax.nn.sigmoid(p[:, FP:2 * FP])
    n = jnp.tanh(p[:, 2 * FP:] - (1.0 - r) * ghn)
    af_nat = afT_all.T
    hnew = (1.0 - z) * n + z * af_nat
    out_ref[...] = jnp.maximum(hnew, 0.0).reshape(M, L, FP)


def kernel(atom_list, bond_list, atom_degree_list, bond_degree_list, atom_mask,
           W_atom, b_atom, W_nb, b_nb, W_align, b_align, W_att, b_att,
           W_ih, W_hh, b_ih, b_hh):
    del atom_mask
    ones = jnp.ones((B, 1, L), jnp.float32)
    bf = jnp.bfloat16
    atomsT2 = jnp.concatenate([atom_list.transpose(0, 2, 1), ones], axis=1).astype(bf)
    bondsT2 = jnp.concatenate([bond_list.transpose(0, 2, 1), ones], axis=1).astype(bf)
    aidxT = atom_degree_list.astype(jnp.int32).transpose(0, 2, 1).astype(bf)
    bidxT = bond_degree_list.astype(jnp.int32).transpose(0, 2, 1).astype(bf)

    w_atom2 = jnp.concatenate([W_atom, b_atom[None, :]], axis=0).astype(bf)
    half_bnb = 0.5 * b_nb[None, :]
    wnb_a2 = jnp.concatenate([W_nb[:F_ATOM], half_bnb], axis=0).astype(bf)
    wnb_b2 = jnp.concatenate([W_nb[F_ATOM:], half_bnb], axis=0).astype(bf)
    w1 = W_align[:FP].astype(bf)
    w2 = W_align[FP:].astype(bf)
    b_align2 = b_align.reshape(1, 1)
    b_att_col = b_att.reshape(FP, 1)
    w_att16 = W_att.astype(bf)
    w_ihT = jnp.concatenate([W_ih.T, W_hh.T], axis=0).astype(bf)
    b_ih_sum = b_ih + b_hh
    w_hhT = W_hh.T[:, 2 * FP:].astype(bf)
    b_hh_n = b_hh[2 * FP:]

    rep = lambda arr: pl.BlockSpec(arr.shape, lambda i: (0,) * arr.ndim)
    row = lambda v: v.reshape(1, -1)

    out = pl.pallas_call(
        _fused_kernel,
        grid=(B // M,),
        in_specs=[
            pl.BlockSpec((M, F_ATOM + 1, L), lambda i: (i, 0, 0)),
            pl.BlockSpec((M, F_BOND + 1, L), lambda i: (i, 0, 0)),
            pl.BlockSpec((M, D, L), lambda i: (i, 0, 0)),
            pl.BlockSpec((M, D, L), lambda i: (i, 0, 0)),
            rep(w_atom2), rep(wnb_a2), rep(wnb_b2),
            rep(w1), rep(w2), rep(b_align2),
            rep(w_att16), rep(b_att_col),
            rep(w_ihT), rep(row(b_ih_sum)),
            rep(w_hhT), rep(row(b_hh_n)),
        ],
        out_specs=pl.BlockSpec((M, L, FP), lambda i: (i, 0, 0)),
        out_shape=jax.ShapeDtypeStruct((B, L, FP), jnp.float32),
        compiler_params=pltpu.CompilerParams(
            dimension_semantics=("parallel",)),
    )(atomsT2, bondsT2, aidxT, bidxT,
      w_atom2, wnb_a2, wnb_b2, w1, w2, b_align2, w_att16, b_att_col,
      w_ihT, row(b_ih_sum), w_hhT, row(b_hh_n))
    return out
